# baseline (speedup 1.0000x reference)
import functools

import jax
import jax.numpy as jnp
from jax import lax
from jax.experimental import pallas as pl
from jax.experimental.pallas import tpu as pltpu
from jax.experimental.pallas import tpu_sc as plsc

_CB = 128
_E = 32
_L = 8
_BATCH = 16384


_BLK_B = 2048


def _quant_body(cb_ref, lat_ref, q_ref):
    cb = cb_ref[...]
    cnorm = jnp.sum(cb * cb, axis=1, keepdims=True)
    for l in range(_L):
        z = lat_ref[l]
        mm = lax.dot_general(
            cb, z, (((1,), (0,)), ((), ())),
            preferred_element_type=jnp.float32)
        zn = jnp.sum(z * z, axis=0, keepdims=True)
        d = (zn + cnorm) - 2.0 * mm
        m = jnp.min(d, axis=0, keepdims=True)
        ii = lax.broadcasted_iota(jnp.int32, d.shape, 0)
        idx = jnp.min(jnp.where(d == m, ii, _CB), axis=0, keepdims=True)
        e = (ii == idx).astype(jnp.float32)
        q_ref[l] = lax.dot_general(
            cb, e, (((0,), (0,)), ((), ())),
            preferred_element_type=jnp.float32)


def _tc_quantize(latent_t, codebook):
    blk = pl.BlockSpec((_L, _E, _BLK_B), lambda i: (0, 0, i))
    return pl.pallas_call(
        _quant_body,
        grid=(_BATCH // _BLK_B,),
        in_specs=[pl.BlockSpec((_CB, _E), lambda i: (0, 0)), blk],
        out_specs=blk,
        out_shape=jax.ShapeDtypeStruct((_L, _E, _BATCH), jnp.float32),
    )(codebook, latent_t)


_NC, _NS = 2, 16
_NW = _NC * _NS


def _sc_dup_body(q_hbm, out, buf, sem):
    wid = lax.axis_index("s") * _NC + lax.axis_index("c")
    l = wid // 4
    r0 = (wid % 4) * 8
    for j in range(2):
        src = q_hbm.at[l, pl.ds(r0 + j * 4, 4)]
        dst = out.at[l, pl.ds(r0 + j * 4, 4)]
        pltpu.async_copy(src, buf, sem).wait()
        pltpu.async_copy(buf, dst, sem).wait()


@functools.cache
def _sc_dup_kernel():
    return pl.kernel(
        _sc_dup_body,
        out_type=jax.ShapeDtypeStruct((_L, _E, _BATCH), jnp.float32),
        mesh=plsc.VectorSubcoreMesh(core_axis_name="c", subcore_axis_name="s"),
        scratch_types=[
            pltpu.VMEM((4, _BATCH), jnp.float32),
            pltpu.SemaphoreType.DMA,
        ],
    )


_FILL = 512
_NDMA = _BATCH // _FILL
_KD = 8


def _bcast_body(cbt_ref, out_ref, buf, sem):
    buf[...] = jnp.broadcast_to(cbt_ref[...][None], (_FILL, _E, _CB))
    for g in range(_NDMA // _KD):
        handles = [
            pltpu.async_copy(
                buf, out_ref.at[pl.ds((g * _KD + t) * _FILL, _FILL)], sem)
            for t in range(_KD)
        ]
        for h in handles:
            h.wait()


def _tc_broadcast(cbt):
    return pl.pallas_call(
        _bcast_body,
        in_specs=[pl.BlockSpec(memory_space=pltpu.VMEM)],
        out_specs=pl.BlockSpec(memory_space=pl.ANY),
        out_shape=jax.ShapeDtypeStruct((_BATCH, _E, _CB), jnp.float32),
        scratch_shapes=[
            pltpu.VMEM((_FILL, _E, _CB), jnp.float32),
            pltpu.SemaphoreType.DMA,
        ],
    )(cbt)


def kernel(latent, codebook):
    latent_t = jnp.transpose(latent, (1, 2, 0))
    q = _tc_quantize(latent_t, codebook)
    p = _sc_dup_kernel()(q)
    cbw = _tc_broadcast(codebook.T)
    policy = jnp.transpose(p, (2, 0, 1))
    quantized = jnp.transpose(q, (2, 0, 1))
    codebook_weight = jnp.swapaxes(cbw, 1, 2)
    return policy, quantized, codebook_weight


# speedup vs baseline: 1.0632x; 1.0632x over previous
import functools

import jax
import jax.numpy as jnp
from jax import lax
from jax.experimental import pallas as pl
from jax.experimental.pallas import tpu as pltpu
from jax.experimental.pallas import tpu_sc as plsc

_CB = 128
_E = 32
_L = 8
_BATCH = 16384
_ROW_F = _CB * _E


_NC, _NS = 2, 16
_NW = _NC * _NS
_BPW = _BATCH // _NW
_BUF = 16
_NST = _BPW // _BUF
_KD = 8


def _sc_bcast_body(cbt_hbm, out, buf, sem):
    wid = lax.axis_index("s") * _NC + lax.axis_index("c")
    base = wid * _BPW * _E
    fills = [
        pltpu.async_copy(cbt_hbm, buf.at[pl.ds(j * _E, _E)], sem)
        for j in range(_BUF)
    ]
    for h in fills:
        h.wait()
    for g in range(_NST // _KD):
        handles = [
            pltpu.async_copy(
                buf,
                out.at[pl.ds(base + (g * _KD + t) * _BUF * _E, _BUF * _E)],
                sem)
            for t in range(_KD)
        ]
        for h in handles:
            h.wait()


@functools.cache
def _sc_bcast_kernel():
    return pl.kernel(
        _sc_bcast_body,
        out_type=jax.ShapeDtypeStruct((_BATCH * _E, _CB), jnp.float32),
        mesh=plsc.VectorSubcoreMesh(core_axis_name="c", subcore_axis_name="s"),
        scratch_types=[
            pltpu.VMEM((_BUF * _E, _CB), jnp.float32),
            pltpu.SemaphoreType.DMA,
        ],
    )


_BLK_B = 2048


def _quant_body(cb_ref, lat_ref, q1_ref, q2_ref):
    cb = cb_ref[...]
    cnorm = jnp.sum(cb * cb, axis=1, keepdims=True)
    for l in range(_L):
        z = lat_ref[l]
        mm = lax.dot_general(
            cb, z, (((1,), (0,)), ((), ())),
            preferred_element_type=jnp.float32)
        zn = jnp.sum(z * z, axis=0, keepdims=True)
        d = (zn + cnorm) - 2.0 * mm
        m = jnp.min(d, axis=0, keepdims=True)
        ii = lax.broadcasted_iota(jnp.int32, d.shape, 0)
        idx = jnp.min(jnp.where(d == m, ii, _CB), axis=0, keepdims=True)
        e = (ii == idx).astype(jnp.float32)
        q = lax.dot_general(cb, e, (((0,), (0,)), ((), ())),
                            preferred_element_type=jnp.float32)
        q1_ref[l] = q
        q2_ref[l] = q


def _tc_quantize(latent_t, codebook):
    out = jax.ShapeDtypeStruct((_L, _E, _BATCH), jnp.float32)
    blk = pl.BlockSpec((_L, _E, _BLK_B), lambda i: (0, 0, i))
    return pl.pallas_call(
        _quant_body,
        grid=(_BATCH // _BLK_B,),
        in_specs=[pl.BlockSpec((_CB, _E), lambda i: (0, 0)), blk],
        out_specs=[blk, blk],
        out_shape=[out, out],
    )(codebook, latent_t)


def kernel(latent, codebook):
    latent_t = jnp.transpose(latent, (1, 2, 0))
    q1, q2 = _tc_quantize(latent_t, codebook)
    policy = jnp.transpose(q1, (2, 0, 1))
    quantized = jnp.transpose(q2, (2, 0, 1))
    cbw = _sc_bcast_kernel()(codebook.T)
    codebook_weight = jnp.swapaxes(cbw.reshape(_BATCH, _E, _CB), 1, 2)
    return policy, quantized, codebook_weight


# speedup vs baseline: 1.0645x; 1.0013x over previous
import functools

import jax
import jax.numpy as jnp
from jax import lax
from jax.experimental import pallas as pl
from jax.experimental.pallas import tpu as pltpu
from jax.experimental.pallas import tpu_sc as plsc

_CB = 128
_E = 32
_L = 8
_BATCH = 16384
_ROW_F = _CB * _E


_NC, _NS = 2, 16
_NW = _NC * _NS
_BPW = _BATCH // _NW
_BUF = 16
_NST = _BPW // _BUF
_KD = 32


def _sc_bcast_body(cbt_hbm, out, buf, sem):
    wid = lax.axis_index("s") * _NC + lax.axis_index("c")
    base = wid * _BPW * _E
    fills = [
        pltpu.async_copy(cbt_hbm, buf.at[pl.ds(j * _E, _E)], sem)
        for j in range(_BUF)
    ]
    for h in fills:
        h.wait()
    for g in range(_NST // _KD):
        handles = [
            pltpu.async_copy(
                buf,
                out.at[pl.ds(base + (g * _KD + t) * _BUF * _E, _BUF * _E)],
                sem)
            for t in range(_KD)
        ]
        for h in handles:
            h.wait()


@functools.cache
def _sc_bcast_kernel():
    return pl.kernel(
        _sc_bcast_body,
        out_type=jax.ShapeDtypeStruct((_BATCH * _E, _CB), jnp.float32),
        mesh=plsc.VectorSubcoreMesh(core_axis_name="c", subcore_axis_name="s"),
        scratch_types=[
            pltpu.VMEM((_BUF * _E, _CB), jnp.float32),
            pltpu.SemaphoreType.DMA,
        ],
    )


_BLK_B = 2048


def _quant_body(cb_ref, lat_ref, q1_ref, q2_ref):
    cb = cb_ref[...]
    cnorm = jnp.sum(cb * cb, axis=1, keepdims=True)
    for l in range(_L):
        z = lat_ref[l]
        mm = lax.dot_general(
            cb, z, (((1,), (0,)), ((), ())),
            preferred_element_type=jnp.float32)
        zn = jnp.sum(z * z, axis=0, keepdims=True)
        d = (zn + cnorm) - 2.0 * mm
        m = jnp.min(d, axis=0, keepdims=True)
        ii = lax.broadcasted_iota(jnp.int32, d.shape, 0)
        idx = jnp.min(jnp.where(d == m, ii, _CB), axis=0, keepdims=True)
        e = (ii == idx).astype(jnp.float32)
        q = lax.dot_general(cb, e, (((0,), (0,)), ((), ())),
                            preferred_element_type=jnp.float32)
        q1_ref[l] = q
        q2_ref[l] = q


def _tc_quantize(latent_t, codebook):
    out = jax.ShapeDtypeStruct((_L, _E, _BATCH), jnp.float32)
    blk = pl.BlockSpec((_L, _E, _BLK_B), lambda i: (0, 0, i))
    return pl.pallas_call(
        _quant_body,
        grid=(_BATCH // _BLK_B,),
        in_specs=[pl.BlockSpec((_CB, _E), lambda i: (0, 0)), blk],
        out_specs=[blk, blk],
        out_shape=[out, out],
    )(codebook, latent_t)


def kernel(latent, codebook):
    latent_t = jnp.transpose(latent, (1, 2, 0))
    q1, q2 = _tc_quantize(latent_t, codebook)
    policy = jnp.transpose(q1, (2, 0, 1))
    quantized = jnp.transpose(q2, (2, 0, 1))
    cbw = _sc_bcast_kernel()(codebook.T)
    codebook_weight = jnp.swapaxes(cbw.reshape(_BATCH, _E, _CB), 1, 2)
    return policy, quantized, codebook_weight
